# Initial kernel scaffold; baseline (speedup 1.0000x reference)
import jax
import jax.numpy as jnp
from jax.experimental import pallas as pl


def kernel(z, edge_index):
    raise NotImplementedError("write your pallas kernel here")


# speedup vs baseline: 1.7887x; 1.7887x over previous
import functools

import jax
import jax.numpy as jnp
from jax import lax
from jax.experimental import pallas as pl
from jax.experimental.pallas import tpu as pltpu
from jax.experimental.pallas import tpu_sc as plsc

D = 256
L = 16
CHUNK = 128
N_WORKERS = 32


def _dot_chunk(srows, drows, outv):
    lane = lax.iota(jnp.int32, L)
    perms = [lane ^ (1 << k) for k in range(4)]
    dnums = lax.GatherDimensionNumbers(
        offset_dims=(), collapsed_slice_dims=(0,), start_index_map=(0,))

    def lane_sum(v):
        for p in perms:
            v = v + lax.gather(v, p[:, None], dnums, slice_sizes=(1,),
                               mode=lax.GatherScatterMode.PROMISE_IN_BOUNDS)
        return v

    def group_body(g, _):
        scores = jnp.zeros((L,), jnp.float32)
        for u in range(L):
            e = g * L + u
            acc = srows[e, pl.ds(0, L)] * drows[e, pl.ds(0, L)]
            for j in range(1, D // L):
                acc = acc + srows[e, pl.ds(j * L, L)] * drows[e, pl.ds(j * L, L)]
            scores = jnp.where(lane == u, lane_sum(acc), scores)
        outv[pl.ds(g * L, L)] = 1.0 / (1.0 + jnp.exp(-scores))
        return _

    lax.fori_loop(0, CHUNK // L, group_body, None)


def kernel(z, edge_index):
    n_edges = edge_index.shape[1]
    n_chunks = n_edges // CHUNK
    chunks_per_w = n_chunks // N_WORKERS
    n_tail = n_chunks - chunks_per_w * N_WORKERS

    mesh = plsc.VectorSubcoreMesh(core_axis_name="c", subcore_axis_name="s")

    @functools.partial(
        pl.kernel,
        mesh=mesh,
        out_type=jax.ShapeDtypeStruct((n_edges,), jnp.float32),
        scratch_types=[
            pltpu.VMEM((CHUNK,), jnp.int32),
            pltpu.VMEM((CHUNK,), jnp.int32),
            pltpu.VMEM((CHUNK, D), jnp.float32),
            pltpu.VMEM((CHUNK, D), jnp.float32),
            pltpu.VMEM((CHUNK,), jnp.float32),
            pltpu.SemaphoreType.DMA,
            pltpu.SemaphoreType.DMA,
        ],
    )
    def sc_kernel(z_hbm, ei_hbm, out_hbm, sidx, didx, srows, drows, outv,
                  sem_s, sem_d):
        wid = lax.axis_index("s") * 2 + lax.axis_index("c")

        def process_chunk(cid):
            base = cid * CHUNK
            pltpu.sync_copy(ei_hbm.at[0, pl.ds(base, CHUNK)], sidx)
            pltpu.sync_copy(ei_hbm.at[1, pl.ds(base, CHUNK)], didx)
            cp_s = pltpu.async_copy(z_hbm.at[sidx], srows, sem_s)
            cp_d = pltpu.async_copy(z_hbm.at[didx], drows, sem_d)
            cp_s.wait()
            cp_d.wait()
            _dot_chunk(srows, drows, outv)
            pltpu.sync_copy(outv, out_hbm.at[pl.ds(base, CHUNK)])

        def chunk_body(i, _):
            process_chunk(i * N_WORKERS + wid)
            return _

        lax.fori_loop(0, chunks_per_w, chunk_body, None)

        @pl.when(wid < n_tail)
        def _():
            process_chunk(chunks_per_w * N_WORKERS + wid)

    return sc_kernel(z, edge_index)


# speedup vs baseline: 2.6735x; 1.4947x over previous
import functools

import jax
import jax.numpy as jnp
from jax import lax
from jax.experimental import pallas as pl
from jax.experimental.pallas import tpu as pltpu
from jax.experimental.pallas import tpu_sc as plsc

D = 256
L = 16
CHUNK = 96
N_WORKERS = 32


def _make_lane_sum():
    lane = lax.iota(jnp.int32, L)
    perms = [lane ^ (1 << k) for k in range(4)]
    dnums = lax.GatherDimensionNumbers(
        offset_dims=(), collapsed_slice_dims=(0,), start_index_map=(0,))

    def lane_sum(v):
        for p in perms:
            v = v + lax.gather(v, p[:, None], dnums, slice_sizes=(1,),
                               mode=lax.GatherScatterMode.PROMISE_IN_BOUNDS)
        return v

    return lane, lane_sum


def _dot_edges(srows, drows, outv, out_base, n_edges):
    lane, lane_sum = _make_lane_sum()
    n_groups = (n_edges + L - 1) // L

    def group(g, _):
        scores = jnp.zeros((L,), jnp.float32)
        for u in range(min(L, n_edges)):
            e = g * L + u
            acc = srows[e, pl.ds(0, L)] * drows[e, pl.ds(0, L)]
            for j in range(1, D // L):
                acc = acc + srows[e, pl.ds(j * L, L)] * drows[e, pl.ds(j * L, L)]
            scores = jnp.where(lane == u, lane_sum(acc), scores)
        outv[pl.ds(out_base + g * L, L)] = 1.0 / (1.0 + jnp.exp(-scores))
        return _

    if n_groups == 1:
        group(0, None)
    else:
        lax.fori_loop(0, n_groups, group, None)


def kernel(z, edge_index):
    n_edges = edge_index.shape[1]
    per_w = n_edges // N_WORKERS
    n_chunks = per_w // CHUNK
    tail = per_w - n_chunks * CHUNK
    n_pairs = n_chunks // 2

    mesh = plsc.VectorSubcoreMesh(core_axis_name="c", subcore_axis_name="s")

    @functools.partial(
        pl.kernel,
        mesh=mesh,
        out_type=jax.ShapeDtypeStruct((n_edges,), jnp.float32),
        scratch_types=[
            pltpu.VMEM((CHUNK,), jnp.int32),
            pltpu.VMEM((CHUNK,), jnp.int32),
            pltpu.VMEM((CHUNK,), jnp.int32),
            pltpu.VMEM((CHUNK,), jnp.int32),
            pltpu.VMEM((CHUNK, D), jnp.float32),
            pltpu.VMEM((CHUNK, D), jnp.float32),
            pltpu.VMEM((CHUNK, D), jnp.float32),
            pltpu.VMEM((CHUNK, D), jnp.float32),
            pltpu.VMEM((tail,), jnp.int32),
            pltpu.VMEM((tail,), jnp.int32),
            pltpu.VMEM((tail, D), jnp.float32),
            pltpu.VMEM((tail, D), jnp.float32),
            pltpu.VMEM((per_w + L,), jnp.float32),
            pltpu.SemaphoreType.DMA,
            pltpu.SemaphoreType.DMA,
            pltpu.SemaphoreType.DMA,
            pltpu.SemaphoreType.DMA,
            pltpu.SemaphoreType.DMA,
            pltpu.SemaphoreType.DMA,
            pltpu.SemaphoreType.DMA,
            pltpu.SemaphoreType.DMA,
        ],
    )
    def sc_kernel(z_hbm, ei_hbm, out_hbm,
                  sidx0, didx0, sidx1, didx1,
                  srows0, drows0, srows1, drows1,
                  tsidx, tdidx, tsrows, tdrows,
                  outv,
                  semS0, semD0, semS1, semD1,
                  isemS0, isemD0, isemS1, isemD1):
        wid = lax.axis_index("s") * 2 + lax.axis_index("c")
        wbase = wid * per_w

        sidx = (sidx0, sidx1)
        didx = (didx0, didx1)
        srows = (srows0, srows1)
        drows = (drows0, drows1)
        semS = (semS0, semS1)
        semD = (semD0, semD1)
        isemS = (isemS0, isemS1)
        isemD = (isemD0, isemD1)

        def chunk_base(i):
            return wbase + jnp.minimum(i, n_chunks - 1) * CHUNK

        def fire_idx(i, p):
            b = chunk_base(i)
            pltpu.async_copy(ei_hbm.at[pl.ds(b, CHUNK)], sidx[p], isemS[p])
            pltpu.async_copy(ei_hbm.at[pl.ds(n_edges + b, CHUNK)], didx[p], isemD[p])

        def wait_idx(p):
            pltpu.make_async_copy(
                ei_hbm.at[pl.ds(0, CHUNK)], sidx[p], isemS[p]).wait()
            pltpu.make_async_copy(
                ei_hbm.at[pl.ds(0, CHUNK)], didx[p], isemD[p]).wait()

        def fire_gather(p):
            pltpu.async_copy(z_hbm.at[sidx[p]], srows[p], semS[p])
            pltpu.async_copy(z_hbm.at[didx[p]], drows[p], semD[p])

        def wait_gather(p):
            pltpu.make_async_copy(z_hbm.at[sidx[p]], srows[p], semS[p]).wait()
            pltpu.make_async_copy(z_hbm.at[didx[p]], drows[p], semD[p]).wait()

        def half(i, p):
            wait_idx(1 - p)
            fire_gather(1 - p)
            fire_idx(i + 2, p)
            wait_gather(p)
            _dot_edges(srows[p], drows[p], outv, i * CHUNK, CHUNK)

        b0 = chunk_base(0)
        pltpu.sync_copy(ei_hbm.at[pl.ds(b0, CHUNK)], sidx0)
        pltpu.sync_copy(ei_hbm.at[pl.ds(n_edges + b0, CHUNK)], didx0)
        fire_gather(0)
        fire_idx(1, 1)

        def pair(j, _):
            half(2 * j, 0)
            half(2 * j + 1, 1)
            return _

        lax.fori_loop(0, n_pairs, pair, None)

        wait_idx(1)
        wait_gather(0)

        tb = wbase + n_chunks * CHUNK
        pltpu.sync_copy(ei_hbm.at[pl.ds(tb, tail)], tsidx)
        pltpu.sync_copy(ei_hbm.at[pl.ds(n_edges + tb, tail)], tdidx)
        cp_s = pltpu.async_copy(z_hbm.at[tsidx], tsrows, semS0)
        cp_d = pltpu.async_copy(z_hbm.at[tdidx], tdrows, semD0)
        cp_s.wait()
        cp_d.wait()
        _dot_edges(tsrows, tdrows, outv, n_chunks * CHUNK, tail)

        pltpu.sync_copy(outv.at[pl.ds(0, per_w)], out_hbm.at[pl.ds(wbase, per_w)])

    return sc_kernel(z, edge_index.reshape(-1))


# speedup vs baseline: 3.9833x; 1.4899x over previous
import functools

import jax
import jax.numpy as jnp
from jax import lax
from jax.experimental import pallas as pl
from jax.experimental.pallas import tpu as pltpu
from jax.experimental.pallas import tpu_sc as plsc

D = 256
L = 16
CHUNK = 96
N_WORKERS = 32


def _make_lane_sum():
    lane = lax.iota(jnp.int32, L)
    perms = [lane ^ (1 << k) for k in range(4)]
    dnums = lax.GatherDimensionNumbers(
        offset_dims=(), collapsed_slice_dims=(0,), start_index_map=(0,))

    def lane_sum(v):
        for p in perms:
            v = v + lax.gather(v, p[:, None], dnums, slice_sizes=(1,),
                               mode=lax.GatherScatterMode.PROMISE_IN_BOUNDS)
        return v

    return lane, lane_sum


_SUB = 4


def _dot_edges(srows, drows, outv, out_base, n_edges):
    lane, lane_sum = _make_lane_sum()
    n_blocks = (n_edges + _SUB - 1) // _SUB
    per_group = L // _SUB

    def block(g, _):
        part = jnp.zeros((L,), jnp.float32)
        off = (g % per_group) * _SUB
        for u in range(min(_SUB, n_edges)):
            e = g * _SUB + u
            acc = srows[e, pl.ds(0, L)] * drows[e, pl.ds(0, L)]
            for j in range(1, D // L):
                acc = acc + srows[e, pl.ds(j * L, L)] * drows[e, pl.ds(j * L, L)]
            part = jnp.where(lane == off + u, lane_sum(acc), part)
        plsc.addupdate(outv.at[pl.ds(out_base + (g // per_group) * L, L)], part)
        return _

    if n_blocks == 1:
        block(0, None)
    else:
        lax.fori_loop(0, n_blocks, block, None)


def kernel(z, edge_index):
    n_edges = edge_index.shape[1]
    per_w = n_edges // N_WORKERS
    n_chunks = per_w // CHUNK
    tail = per_w - n_chunks * CHUNK
    n_pairs = n_chunks // 2

    mesh = plsc.VectorSubcoreMesh(core_axis_name="c", subcore_axis_name="s")

    @functools.partial(
        pl.kernel,
        mesh=mesh,
        out_type=jax.ShapeDtypeStruct((n_edges,), jnp.float32),
        scratch_types=[
            pltpu.VMEM((CHUNK,), jnp.int32),
            pltpu.VMEM((CHUNK,), jnp.int32),
            pltpu.VMEM((CHUNK,), jnp.int32),
            pltpu.VMEM((CHUNK,), jnp.int32),
            pltpu.VMEM((CHUNK, D), jnp.float32),
            pltpu.VMEM((CHUNK, D), jnp.float32),
            pltpu.VMEM((CHUNK, D), jnp.float32),
            pltpu.VMEM((CHUNK, D), jnp.float32),
            pltpu.VMEM((tail,), jnp.int32),
            pltpu.VMEM((tail,), jnp.int32),
            pltpu.VMEM((tail, D), jnp.float32),
            pltpu.VMEM((tail, D), jnp.float32),
            pltpu.VMEM((((per_w + L - 1) // L) * L,), jnp.float32),
            pltpu.SemaphoreType.DMA,
            pltpu.SemaphoreType.DMA,
            pltpu.SemaphoreType.DMA,
            pltpu.SemaphoreType.DMA,
            pltpu.SemaphoreType.DMA,
            pltpu.SemaphoreType.DMA,
            pltpu.SemaphoreType.DMA,
            pltpu.SemaphoreType.DMA,
        ],
    )
    def sc_kernel(z_hbm, ei_hbm, out_hbm,
                  sidx0, didx0, sidx1, didx1,
                  srows0, drows0, srows1, drows1,
                  tsidx, tdidx, tsrows, tdrows,
                  outv,
                  semS0, semD0, semS1, semD1,
                  isemS0, isemD0, isemS1, isemD1):
        wid = lax.axis_index("s") * 2 + lax.axis_index("c")
        wbase = wid * per_w

        sidx = (sidx0, sidx1)
        didx = (didx0, didx1)
        srows = (srows0, srows1)
        drows = (drows0, drows1)
        semS = (semS0, semS1)
        semD = (semD0, semD1)
        isemS = (isemS0, isemS1)
        isemD = (isemD0, isemD1)

        def chunk_base(i):
            return wbase + jnp.minimum(i, n_chunks - 1) * CHUNK

        def fire_idx(i, p):
            b = chunk_base(i)
            pltpu.async_copy(ei_hbm.at[pl.ds(b, CHUNK)], sidx[p], isemS[p])
            pltpu.async_copy(ei_hbm.at[pl.ds(n_edges + b, CHUNK)], didx[p], isemD[p])

        def wait_idx(p):
            pltpu.make_async_copy(
                ei_hbm.at[pl.ds(0, CHUNK)], sidx[p], isemS[p]).wait()
            pltpu.make_async_copy(
                ei_hbm.at[pl.ds(0, CHUNK)], didx[p], isemD[p]).wait()

        def fire_gather(p):
            pltpu.async_copy(z_hbm.at[sidx[p]], srows[p], semS[p])
            pltpu.async_copy(z_hbm.at[didx[p]], drows[p], semD[p])

        def wait_gather(p):
            pltpu.make_async_copy(z_hbm.at[sidx[p]], srows[p], semS[p]).wait()
            pltpu.make_async_copy(z_hbm.at[didx[p]], drows[p], semD[p]).wait()

        def half(i, p):
            wait_idx(1 - p)
            fire_gather(1 - p)
            fire_idx(i + 2, p)
            wait_gather(p)
            _dot_edges(srows[p], drows[p], outv, i * CHUNK, CHUNK)

        n_groups = ((per_w + L - 1) // L)
        zeros = jnp.zeros((L,), jnp.float32)

        def zero_body(i, _):
            outv[pl.ds(i * L, L)] = zeros
            return _

        lax.fori_loop(0, n_groups, zero_body, None)

        b0 = chunk_base(0)
        pltpu.sync_copy(ei_hbm.at[pl.ds(b0, CHUNK)], sidx0)
        pltpu.sync_copy(ei_hbm.at[pl.ds(n_edges + b0, CHUNK)], didx0)
        fire_gather(0)
        fire_idx(1, 1)

        def pair(j, _):
            half(2 * j, 0)
            half(2 * j + 1, 1)
            return _

        lax.fori_loop(0, n_pairs, pair, None)

        wait_idx(1)
        wait_gather(0)

        tb = wbase + n_chunks * CHUNK
        pltpu.sync_copy(ei_hbm.at[pl.ds(tb, tail)], tsidx)
        pltpu.sync_copy(ei_hbm.at[pl.ds(n_edges + tb, tail)], tdidx)
        cp_s = pltpu.async_copy(z_hbm.at[tsidx], tsrows, semS0)
        cp_d = pltpu.async_copy(z_hbm.at[tdidx], tdrows, semD0)
        cp_s.wait()
        cp_d.wait()
        _dot_edges(tsrows, tdrows, outv, n_chunks * CHUNK, tail)

        def sig_body(i, _):
            v = outv[pl.ds(i * L, L)]
            outv[pl.ds(i * L, L)] = 1.0 / (1.0 + jnp.exp(-v))
            return _

        lax.fori_loop(0, n_groups, sig_body, None)

        pltpu.sync_copy(outv.at[pl.ds(0, per_w)], out_hbm.at[pl.ds(wbase, per_w)])

    return sc_kernel(z, edge_index.reshape(-1))


# speedup vs baseline: 5.0262x; 1.2618x over previous
import functools

import jax
import jax.numpy as jnp
from jax import lax
from jax.experimental import pallas as pl
from jax.experimental.pallas import tpu as pltpu
from jax.experimental.pallas import tpu_sc as plsc

D = 256
L = 16
CHUNK = 96
N_WORKERS = 32


def _make_lane_sum():
    lane = lax.iota(jnp.int32, L)
    perms = [lane ^ (1 << k) for k in range(4)]
    dnums = lax.GatherDimensionNumbers(
        offset_dims=(), collapsed_slice_dims=(0,), start_index_map=(0,))

    def lane_sum(v):
        for p in perms:
            v = v + lax.gather(v, p[:, None], dnums, slice_sizes=(1,),
                               mode=lax.GatherScatterMode.PROMISE_IN_BOUNDS)
        return v

    return lane, lane_sum


_SUB = 2


def _dot_edges(srows, drows, outv, out_base, n_edges):
    lane, lane_sum = _make_lane_sum()
    n_blocks = (n_edges + _SUB - 1) // _SUB
    per_group = L // _SUB

    def block(g, _):
        part = jnp.zeros((L,), jnp.float32)
        off = (g % per_group) * _SUB
        for u in range(min(_SUB, n_edges)):
            e = g * _SUB + u
            acc0 = srows[e, pl.ds(0, L)] * drows[e, pl.ds(0, L)]
            acc1 = srows[e, pl.ds(L, L)] * drows[e, pl.ds(L, L)]
            for j in range(2, D // L, 2):
                acc0 = acc0 + srows[e, pl.ds(j * L, L)] * drows[e, pl.ds(j * L, L)]
                acc1 = acc1 + srows[e, pl.ds((j + 1) * L, L)] * drows[e, pl.ds((j + 1) * L, L)]
            part = jnp.where(lane == off + u, lane_sum(acc0 + acc1), part)
        plsc.addupdate(outv.at[pl.ds(out_base + (g // per_group) * L, L)], part)
        return _

    if n_blocks == 1:
        block(0, None)
    else:
        lax.fori_loop(0, n_blocks, block, None)


def kernel(z, edge_index):
    n_edges = edge_index.shape[1]
    per_w = n_edges // N_WORKERS
    n_chunks = per_w // CHUNK
    tail = per_w - n_chunks * CHUNK
    n_pairs = n_chunks // 2

    mesh = plsc.VectorSubcoreMesh(core_axis_name="c", subcore_axis_name="s")

    @functools.partial(
        pl.kernel,
        mesh=mesh,
        out_type=jax.ShapeDtypeStruct((n_edges,), jnp.float32),
        scratch_types=[
            pltpu.VMEM((CHUNK,), jnp.int32),
            pltpu.VMEM((CHUNK,), jnp.int32),
            pltpu.VMEM((CHUNK,), jnp.int32),
            pltpu.VMEM((CHUNK,), jnp.int32),
            pltpu.VMEM((CHUNK, D), jnp.float32),
            pltpu.VMEM((CHUNK, D), jnp.float32),
            pltpu.VMEM((CHUNK, D), jnp.float32),
            pltpu.VMEM((CHUNK, D), jnp.float32),
            pltpu.VMEM((tail,), jnp.int32),
            pltpu.VMEM((tail,), jnp.int32),
            pltpu.VMEM((tail, D), jnp.float32),
            pltpu.VMEM((tail, D), jnp.float32),
            pltpu.VMEM((((per_w + L - 1) // L) * L,), jnp.float32),
            pltpu.SemaphoreType.DMA,
            pltpu.SemaphoreType.DMA,
            pltpu.SemaphoreType.DMA,
            pltpu.SemaphoreType.DMA,
            pltpu.SemaphoreType.DMA,
            pltpu.SemaphoreType.DMA,
            pltpu.SemaphoreType.DMA,
            pltpu.SemaphoreType.DMA,
        ],
    )
    def sc_kernel(z_hbm, ei_hbm, out_hbm,
                  sidx0, didx0, sidx1, didx1,
                  srows0, drows0, srows1, drows1,
                  tsidx, tdidx, tsrows, tdrows,
                  outv,
                  semS0, semD0, semS1, semD1,
                  isemS0, isemD0, isemS1, isemD1):
        wid = lax.axis_index("s") * 2 + lax.axis_index("c")
        wbase = wid * per_w

        sidx = (sidx0, sidx1)
        didx = (didx0, didx1)
        srows = (srows0, srows1)
        drows = (drows0, drows1)
        semS = (semS0, semS1)
        semD = (semD0, semD1)
        isemS = (isemS0, isemS1)
        isemD = (isemD0, isemD1)

        def chunk_base(i):
            return wbase + jnp.minimum(i, n_chunks - 1) * CHUNK

        def fire_idx(i, p):
            b = chunk_base(i)
            pltpu.async_copy(ei_hbm.at[pl.ds(b, CHUNK)], sidx[p], isemS[p])
            pltpu.async_copy(ei_hbm.at[pl.ds(n_edges + b, CHUNK)], didx[p], isemD[p])

        def wait_idx(p):
            pltpu.make_async_copy(
                ei_hbm.at[pl.ds(0, CHUNK)], sidx[p], isemS[p]).wait()
            pltpu.make_async_copy(
                ei_hbm.at[pl.ds(0, CHUNK)], didx[p], isemD[p]).wait()

        def fire_gather(p):
            pltpu.async_copy(z_hbm.at[sidx[p]], srows[p], semS[p])
            pltpu.async_copy(z_hbm.at[didx[p]], drows[p], semD[p])

        def wait_gather(p):
            pltpu.make_async_copy(z_hbm.at[sidx[p]], srows[p], semS[p]).wait()
            pltpu.make_async_copy(z_hbm.at[didx[p]], drows[p], semD[p]).wait()

        def half(i, p):
            wait_idx(1 - p)
            fire_gather(1 - p)
            fire_idx(i + 2, p)
            wait_gather(p)
            _dot_edges(srows[p], drows[p], outv, i * CHUNK, CHUNK)

        n_groups = ((per_w + L - 1) // L)
        zeros = jnp.zeros((L,), jnp.float32)

        def zero_body(i, _):
            outv[pl.ds(i * L, L)] = zeros
            return _

        lax.fori_loop(0, n_groups, zero_body, None)

        b0 = chunk_base(0)
        pltpu.sync_copy(ei_hbm.at[pl.ds(b0, CHUNK)], sidx0)
        pltpu.sync_copy(ei_hbm.at[pl.ds(n_edges + b0, CHUNK)], didx0)
        fire_gather(0)
        fire_idx(1, 1)

        def pair(j, _):
            half(2 * j, 0)
            half(2 * j + 1, 1)
            return _

        lax.fori_loop(0, n_pairs, pair, None)

        wait_idx(1)
        wait_gather(0)

        tb = wbase + n_chunks * CHUNK
        pltpu.sync_copy(ei_hbm.at[pl.ds(tb, tail)], tsidx)
        pltpu.sync_copy(ei_hbm.at[pl.ds(n_edges + tb, tail)], tdidx)
        cp_s = pltpu.async_copy(z_hbm.at[tsidx], tsrows, semS0)
        cp_d = pltpu.async_copy(z_hbm.at[tdidx], tdrows, semD0)
        cp_s.wait()
        cp_d.wait()
        _dot_edges(tsrows, tdrows, outv, n_chunks * CHUNK, tail)

        def sig_body(i, _):
            v = outv[pl.ds(i * L, L)]
            outv[pl.ds(i * L, L)] = 1.0 / (1.0 + jnp.exp(-v))
            return _

        lax.fori_loop(0, n_groups, sig_body, None)

        pltpu.sync_copy(outv.at[pl.ds(0, per_w)], out_hbm.at[pl.ds(wbase, per_w)])

    return sc_kernel(z, edge_index.reshape(-1))
